# baseline (speedup 1.0000x reference)
import functools

import jax
import jax.numpy as jnp
from jax import lax
from jax.experimental import pallas as pl
from jax.experimental.pallas import tpu as pltpu
from jax.experimental.pallas import tpu_sc as plsc

_B, _N, _F, _D = 256, 512, 9, 64
_BN = _B * _N
_NC, _NS = 2, 16
_NW = _NC * _NS
_NPW = _N // _NW
_C = 64
_CHUNKS = _B // _C
_T = _NPW * _CHUNKS
_ROWS_OUT = _N + 1
_BP = _B + 1


def _body(deg_max_in, deg_max_out,
          x_ref, ind_ref, outd_ref, atom_ref, int_ref, outt_ref, gt_ref,
          out_ref,
          gx, gdi, gdo,
          arows0, irows0, orows0,
          arows1, irows1, orows1,
          plane, gt_v, tok_v,
          ssem, gsem0, gsem1, psem):
    wid = lax.axis_index("s") * _NC + lax.axis_index("c")
    n0 = wid * _NPW

    arows = (arows0, arows1)
    irows = (irows0, irows1)
    orows = (orows0, orows1)
    gsem = (gsem0, gsem1)
    iota16 = lax.iota(jnp.int32, 16)

    def stage(nl):
        n = n0 + nl
        q = nl % 2
        for f in range(_F):
            pltpu.async_copy(x_ref.at[pl.ds(f * _BN + n * _B, _B)],
                             gx.at[q, pl.ds(f * _B, _B)], ssem)
        pltpu.async_copy(ind_ref.at[pl.ds(n * _B, _B)], gdi.at[q], ssem)
        pltpu.async_copy(outd_ref.at[pl.ds(n * _B, _B)], gdo.at[q], ssem)

    def stage_wait_clip(q):
        for f in range(_F):
            pltpu.make_async_copy(x_ref.at[pl.ds(0, _B)],
                                  gx.at[0, pl.ds(0, _B)], ssem).wait()
        pltpu.make_async_copy(ind_ref.at[pl.ds(0, _B)], gdi.at[0],
                              ssem).wait()
        pltpu.make_async_copy(outd_ref.at[pl.ds(0, _B)], gdo.at[0],
                              ssem).wait()
        for j in range(_B // 16):
            s = pl.ds(j * 16, 16)
            gdi[q, s] = jnp.minimum(jnp.maximum(gdi[q, s], 0), deg_max_in)
            gdo[q, s] = jnp.minimum(jnp.maximum(gdo[q, s], 0), deg_max_out)

    def fire_t(t, p, *, pre_wait):
        nl = t // _CHUNKS
        cb = t % _CHUNKS
        q = nl % 2
        if pre_wait:
            @pl.when(cb == 0)
            def _():
                stage_wait_clip(q)

        for f in range(_F):
            pltpu.async_copy(
                atom_ref.at[gx.at[q, pl.ds(f * _B + cb * _C, _C)]],
                arows[p].at[pl.ds(f * _C, _C), :], gsem[p])
        pltpu.async_copy(int_ref.at[gdi.at[q, pl.ds(cb * _C, _C)]],
                         irows[p], gsem[p])
        pltpu.async_copy(outt_ref.at[gdo.at[q, pl.ds(cb * _C, _C)]],
                         orows[p], gsem[p])

    def gather_wait(p):
        for f in range(_F):
            pltpu.make_async_copy(atom_ref.at[gx.at[0, pl.ds(0, _C)]],
                                  arows[p].at[pl.ds(f * _C, _C), :],
                                  gsem[p]).wait()
        pltpu.make_async_copy(int_ref.at[gdi.at[0, pl.ds(0, _C)]],
                              irows[p], gsem[p]).wait()
        pltpu.make_async_copy(outt_ref.at[gdo.at[0, pl.ds(0, _C)]],
                              orows[p], gsem[p]).wait()

    def plane_drain():
        pltpu.make_async_copy(plane.at[0, :, pl.ds(0, _B)],
                              out_ref.at[0, :, :], psem).wait()

    def compute_store(t, p):
        nl = t // _CHUNKS
        cb = t % _CHUNKS
        qq = nl % 2
        ar, ir, orr = arows[p], irows[p], orows[p]
        qv = jnp.full((16,), qq, jnp.int32)

        @pl.when(jnp.logical_and(cb == 1, nl >= 1))
        def _():
            plane_drain()

        def node_body(bc, carry):
            colv = jnp.full((16,), cb * _C + bc, jnp.int32)
            for dj in range(_D // 16):
                s = pl.ds(dj * 16, 16)
                acc = ir[bc, s] + orr[bc, s]
                for f in range(_F):
                    acc = acc + ar[f * _C + bc, s]
                plsc.store_scatter(plane, [qv, iota16 + (dj * 16), colv],
                                   acc)
            return carry

        lax.fori_loop(0, _C, node_body, 0)

        @pl.when(cb == _CHUNKS - 1)
        def _():
            pltpu.async_copy(plane.at[qq, :, pl.ds(0, _B)],
                             out_ref.at[1 + n0 + nl, :, :], psem)

    pltpu.sync_copy(gt_ref, gt_v)
    dbase = 2 * wid
    for r in range(2):
        dv = jnp.full((16,), dbase + r, jnp.int32)
        vec = plsc.load_gather(gt_v, [dv])
        for j in range(_B // 16):
            tok_v[r, pl.ds(j * 16, 16)] = vec
    pltpu.sync_copy(tok_v, out_ref.at[0, pl.ds(dbase, 2), :])

    stage(0)
    stage_wait_clip(0)
    stage(1)
    fire_t(0, 0, pre_wait=False)

    def pair_body(i, carry):
        t0 = 2 * i
        t1 = t0 + 1

        fire_t(t1, 1, pre_wait=False)
        gather_wait(0)
        compute_store(t0, 0)

        @pl.when(t0 + 2 < _T)
        def _():
            fire_t(t0 + 2, 0, pre_wait=True)

        gather_wait(1)
        compute_store(t1, 1)

        nl = t1 // _CHUNKS

        @pl.when(jnp.logical_and((t1 % _CHUNKS) == (_CHUNKS - 1),
                                 nl + 2 < _NPW))
        def _():
            stage(nl + 2)

        return carry

    lax.fori_loop(0, _T // 2, pair_body, 0)
    plane_drain()


@jax.jit
def _run(x_t, ind_t, outd_t, atom_table, in_table, out_table, graph_token):
    mesh = plsc.VectorSubcoreMesh(core_axis_name="c", subcore_axis_name="s")
    body = functools.partial(_body, in_table.shape[0] - 1,
                             out_table.shape[0] - 1)
    buf_types = [
        pltpu.VMEM((_F * _C, _D), jnp.float32),
        pltpu.VMEM((_C, _D), jnp.float32),
        pltpu.VMEM((_C, _D), jnp.float32),
    ]
    return pl.kernel(
        body,
        out_type=jax.ShapeDtypeStruct((_ROWS_OUT, _D, _B), jnp.float32),
        mesh=mesh,
        compiler_params=pltpu.CompilerParams(use_tc_tiling_on_sc=False,
                                             needs_layout_passes=False),
        scratch_types=([
            pltpu.VMEM((2, _F * _B), jnp.int32),
            pltpu.VMEM((2, _B), jnp.int32),
            pltpu.VMEM((2, _B), jnp.int32),
        ] + buf_types + buf_types + [
            pltpu.VMEM((2, _D, _BP), jnp.float32),
            pltpu.VMEM((_D,), jnp.float32),
            pltpu.VMEM((2, _B), jnp.float32),
            pltpu.SemaphoreType.DMA,
            pltpu.SemaphoreType.DMA,
            pltpu.SemaphoreType.DMA,
            pltpu.SemaphoreType.DMA,
        ]),
    )(x_t, ind_t, outd_t, atom_table, in_table, out_table, graph_token)


def kernel(x, in_degree, out_degree, atom_table, in_table, out_table,
           graph_token):
    x_t = x.transpose(2, 1, 0).reshape(-1).astype(jnp.int32)
    ind_t = in_degree.transpose(1, 0).reshape(-1).astype(jnp.int32)
    outd_t = out_degree.transpose(1, 0).reshape(-1).astype(jnp.int32)
    out_k = _run(x_t, ind_t, outd_t, atom_table.astype(jnp.float32),
                 in_table.astype(jnp.float32), out_table.astype(jnp.float32),
                 graph_token.reshape(-1).astype(jnp.float32))
    return out_k.transpose(2, 0, 1)


# speedup vs baseline: 1.0988x; 1.0988x over previous
import functools

import jax
import jax.numpy as jnp
from jax import lax
from jax.experimental import pallas as pl
from jax.experimental.pallas import tpu as pltpu
from jax.experimental.pallas import tpu_sc as plsc

_B, _N, _F, _D = 256, 512, 9, 64
_BN = _B * _N
_NC, _NS = 2, 16
_NW = _NC * _NS
_GPW = _B // _NW
_C = 64
_CHUNKS = _N // _C
_T = _GPW * _CHUNKS
_ROWS_OUT = _N + 1


def _body(deg_max_in, deg_max_out,
          x_ref, ind_ref, outd_ref, atom_ref, int_ref, outt_ref, gt_ref,
          out_ref,
          gx0, gdi0, gdo0, gx1, gdi1, gdo1,
          arows0, irows0, orows0, obuf0,
          arows1, irows1, orows1, obuf1,
          gt_v, ssem0, ssem1, gsem0, gsem1, osem0, osem1):
    wid = lax.axis_index("s") * _NC + lax.axis_index("c")
    graph0 = wid * _GPW

    gx = (gx0, gx1)
    gdi = (gdi0, gdi1)
    gdo = (gdo0, gdo1)
    ssem = (ssem0, ssem1)
    arows = (arows0, arows1)
    irows = (irows0, irows1)
    orows = (orows0, orows1)
    obuf = (obuf0, obuf1)
    gsem = (gsem0, gsem1)
    osem = (osem0, osem1)

    def stage(g, q):
        b = graph0 + g
        for f in range(_F):
            pltpu.async_copy(x_ref.at[pl.ds(f * _BN + b * _N, _N)],
                             gx[q].at[pl.ds(f * _N, _N)], ssem[q])
        pltpu.async_copy(ind_ref.at[pl.ds(b * _N, _N)], gdi[q], ssem[q])
        pltpu.async_copy(outd_ref.at[pl.ds(b * _N, _N)], gdo[q], ssem[q])

    def stage_wait_clip(q):
        for f in range(_F):
            pltpu.make_async_copy(x_ref.at[pl.ds(0, _N)],
                                  gx[q].at[pl.ds(0, _N)], ssem[q]).wait()
        pltpu.make_async_copy(ind_ref.at[pl.ds(0, _N)], gdi[q],
                              ssem[q]).wait()
        pltpu.make_async_copy(outd_ref.at[pl.ds(0, _N)], gdo[q],
                              ssem[q]).wait()
        for j in range(_N // 16):
            s = pl.ds(j * 16, 16)
            gdi[q][s] = jnp.minimum(jnp.maximum(gdi[q][s], 0), deg_max_in)
            gdo[q][s] = jnp.minimum(jnp.maximum(gdo[q][s], 0), deg_max_out)

    def fire(ch, q, p):
        for f in range(_F):
            pltpu.async_copy(
                atom_ref.at[gx[q].at[pl.ds(f * _N + ch * _C, _C)]],
                arows[p].at[pl.ds(f * _C, _C), :], gsem[p])
        pltpu.async_copy(int_ref.at[gdi[q].at[pl.ds(ch * _C, _C)]],
                         irows[p], gsem[p])
        pltpu.async_copy(outt_ref.at[gdo[q].at[pl.ds(ch * _C, _C)]],
                         orows[p], gsem[p])

    def gather_wait(p):
        for f in range(_F):
            pltpu.make_async_copy(atom_ref.at[gx[0].at[pl.ds(0, _C)]],
                                  arows[p].at[pl.ds(f * _C, _C), :],
                                  gsem[p]).wait()
        pltpu.make_async_copy(int_ref.at[gdi[0].at[pl.ds(0, _C)]],
                              irows[p], gsem[p]).wait()
        pltpu.make_async_copy(outt_ref.at[gdo[0].at[pl.ds(0, _C)]],
                              orows[p], gsem[p]).wait()

    def store_drain(p):
        pltpu.make_async_copy(obuf[p], out_ref.at[0, pl.ds(1, _C), :],
                              osem[p]).wait()

    def compute_store(g, ch, p):
        ar, ir, orr, ob = arows[p], irows[p], orows[p], obuf[p]

        def node_body(c, carry):
            for dj in range(_D // 16):
                s = pl.ds(dj * 16, 16)
                acc = ir[c, s] + orr[c, s]
                for f in range(_F):
                    acc = acc + ar[f * _C + c, s]
                ob[c, s] = acc
            return carry

        lax.fori_loop(0, _C, node_body, 0)
        pltpu.async_copy(ob, out_ref.at[graph0 + g, pl.ds(1 + ch * _C, _C), :],
                         osem[p])

    pltpu.sync_copy(gt_ref, gt_v)
    for g in range(_GPW):
        pltpu.sync_copy(gt_v, out_ref.at[graph0 + g, pl.ds(0, 1), :])

    stage(0, 0)
    stage_wait_clip(0)
    stage(1, 1)
    fire(0, 0, 0)

    for g in range(_GPW):
        q = g % 2

        def pair_body(i2, carry, g=g, q=q):
            ch0 = 2 * i2
            fire(ch0 + 1, q, 1)
            gather_wait(0)
            if g == 0:
                @pl.when(i2 >= 1)
                def _():
                    store_drain(0)
            else:
                store_drain(0)
            compute_store(g, ch0, 0)
            fire(ch0 + 2, q, 0)
            gather_wait(1)
            if g == 0:
                @pl.when(i2 >= 1)
                def _():
                    store_drain(1)
            else:
                store_drain(1)
            compute_store(g, ch0 + 1, 1)
            return carry

        lax.fori_loop(0, _CHUNKS // 2 - 1, pair_body, 0)

        fire(_CHUNKS - 1, q, 1)
        gather_wait(0)
        store_drain(0)
        compute_store(g, _CHUNKS - 2, 0)
        if g + 1 < _GPW:
            stage_wait_clip(1 - q)
            fire(0, 1 - q, 0)
        gather_wait(1)
        store_drain(1)
        compute_store(g, _CHUNKS - 1, 1)
        if g + 2 < _GPW:
            stage(g + 2, q)

    store_drain(0)
    store_drain(1)


@jax.jit
def _run(x_fm, ind, outd, atom_table, in_table, out_table, graph_token):
    mesh = plsc.VectorSubcoreMesh(core_axis_name="c", subcore_axis_name="s")
    body = functools.partial(_body, in_table.shape[0] - 1,
                             out_table.shape[0] - 1)
    stage_types = [
        pltpu.VMEM((_F * _N,), jnp.int32),
        pltpu.VMEM((_N,), jnp.int32),
        pltpu.VMEM((_N,), jnp.int32),
    ]
    buf_types = [
        pltpu.VMEM((_F * _C, _D), jnp.float32),
        pltpu.VMEM((_C, _D), jnp.float32),
        pltpu.VMEM((_C, _D), jnp.float32),
        pltpu.VMEM((_C, _D), jnp.float32),
    ]
    return pl.kernel(
        body,
        out_type=jax.ShapeDtypeStruct((_B, _ROWS_OUT, _D), jnp.float32),
        mesh=mesh,
        compiler_params=pltpu.CompilerParams(use_tc_tiling_on_sc=False),
        scratch_types=stage_types + stage_types + buf_types + buf_types + [
            pltpu.VMEM((1, _D), jnp.float32),
            pltpu.SemaphoreType.DMA,
            pltpu.SemaphoreType.DMA,
            pltpu.SemaphoreType.DMA,
            pltpu.SemaphoreType.DMA,
            pltpu.SemaphoreType.DMA,
            pltpu.SemaphoreType.DMA,
        ],
    )(x_fm, ind, outd, atom_table, in_table, out_table, graph_token)


def kernel(x, in_degree, out_degree, atom_table, in_table, out_table,
           graph_token):
    x_fm = x.transpose(2, 0, 1).reshape(-1).astype(jnp.int32)
    ind = in_degree.reshape(-1).astype(jnp.int32)
    outd = out_degree.reshape(-1).astype(jnp.int32)
    return _run(x_fm, ind, outd, atom_table.astype(jnp.float32),
                in_table.astype(jnp.float32), out_table.astype(jnp.float32),
                graph_token.astype(jnp.float32))
